# baseline (speedup 1.0000x reference)
import functools

import jax
import jax.numpy as jnp
from jax import lax
from jax.experimental import pallas as pl
from jax.experimental.pallas import tpu as pltpu
from jax.experimental.pallas import tpu_sc as plsc

P = 3
NBATCH = 1024
F = 128
NH = 8
HEAD_IN = F // NH
OUT_DIM = 64
OUT_SZ = OUT_DIM // NH
HID = 128


def _attn_body(xg_ref, A_ref, bias_ref, W1_ref, b1_ref, W2_ref, b2_ref,
               Wf_ref, bf_ref, a1_ref, a2_ref, Wm_ref, bm_ref,
               out_ref):
    x = xg_ref[0]
    A = A_ref[0]
    bias = bias_ref[0]

    deg = jnp.maximum(jnp.sum(A, axis=1, keepdims=True), 1.0)
    agg1 = jnp.dot(A, x, preferred_element_type=jnp.float32) / deg
    h = jax.nn.relu(jnp.dot(agg1, W1_ref[0], preferred_element_type=jnp.float32)
                    + b1_ref[0])
    agg2 = jnp.dot(A, h, preferred_element_type=jnp.float32) / deg
    fe = jnp.dot(agg2, W2_ref[0], preferred_element_type=jnp.float32) + b2_ref[0]

    attns = []
    for nh in range(NH):
        xh = fe[:, nh * HEAD_IN:(nh + 1) * HEAD_IN]
        f = jnp.dot(xh, Wf_ref[0, nh], preferred_element_type=jnp.float32) \
            + bf_ref[0, nh]
        f1 = jnp.dot(f, a1_ref[0, nh].reshape(OUT_SZ, 1),
                     preferred_element_type=jnp.float32)
        f2 = jnp.dot(f, a2_ref[0, nh].reshape(OUT_SZ, 1),
                     preferred_element_type=jnp.float32)
        logits = f1 + f2.T
        z = jnp.where(logits >= 0.0, logits, 0.2 * logits) + bias
        m = jnp.max(z, axis=1, keepdims=True)
        e = jnp.exp(z - m)
        s = jnp.sum(e, axis=1, keepdims=True)
        coefs = e / s
        av = jnp.dot(coefs, f, preferred_element_type=jnp.float32)
        attns.append(jnp.where(av > 0.0, av, jnp.exp(av) - 1.0))
    h_1 = jnp.concatenate(attns, axis=-1)
    h1t = jnp.dot(h_1, Wm_ref[...], preferred_element_type=jnp.float32) \
        + bm_ref[...]
    out_ref[...] = h1t


def _attn_call(i, xg, A, bias, W1, b1, W2, b2, Wf, bf, a1, a2, Wm, bm):
    bs_meta3 = lambda shp: pl.BlockSpec((1,) + shp,
                                        lambda g, i=i: (i,) + (0,) * len(shp))
    bs_full = lambda shp: pl.BlockSpec(shp, lambda g: (0,) * len(shp))
    return pl.pallas_call(
        _attn_body,
        grid=(1,),
        in_specs=[
            bs_meta3((NBATCH, F)),
            bs_meta3((NBATCH, NBATCH)),
            bs_meta3((NBATCH, NBATCH)),
            bs_meta3((F, HID)),
            bs_meta3((1, HID)),
            bs_meta3((HID, F)),
            bs_meta3((1, F)),
            bs_meta3((NH, HEAD_IN, OUT_SZ)),
            bs_meta3((NH, 1, OUT_SZ)),
            bs_meta3((NH, OUT_SZ)),
            bs_meta3((NH, OUT_SZ)),
            bs_full((OUT_DIM, OUT_DIM)),
            bs_full((1, OUT_DIM)),
        ],
        out_specs=pl.BlockSpec((NBATCH, OUT_DIM), lambda g: (0, 0)),
        out_shape=jax.ShapeDtypeStruct((NBATCH, OUT_DIM), jnp.float32),
    )(xg, A, bias, W1, b1[:, None, :], W2, b2[:, None, :], Wf,
      bf[:, :, None, :], a1, a2, Wm, bm[None])


def _sem_body(m0_ref, m1_ref, m2_ref, wom_ref, bom_ref, uom_ref, out_ref):
    ms = [m0_ref[...], m1_ref[...], m2_ref[...]]
    vus = []
    for j in range(P):
        v = jnp.tanh(jnp.dot(ms[j], wom_ref[...],
                             preferred_element_type=jnp.float32)
                     + bom_ref[...])
        vu = jnp.dot(v, uom_ref[...].reshape(HID, 1),
                     preferred_element_type=jnp.float32)
        vus.append(vu)
    vu_all = jnp.concatenate(vus, axis=-1)
    mx = jnp.max(vu_all, axis=1, keepdims=True)
    ev = jnp.exp(vu_all - mx)
    al = ev / jnp.sum(ev, axis=1, keepdims=True)
    acc = al[:, 0:1] * ms[0]
    for j in range(1, P):
        acc = acc + al[:, j:j + 1] * ms[j]
    out_ref[...] = acc


def _sem_call(m0, m1, m2, w_omega, b_omega, u_omega):
    return pl.pallas_call(
        _sem_body,
        out_shape=jax.ShapeDtypeStruct((NBATCH, OUT_DIM), jnp.float32),
    )(m0, m1, m2, w_omega, b_omega[None], u_omega[None])


@jax.jit
def _dense_call(xg, A, bias, W1, b1, W2, b2,
                Wf, bf, a1, a2, Wm, bm, w_omega, b_omega, u_omega):
    hs = []
    for i in range(P):
        hs.append(_attn_call(i, xg, A, bias, W1, b1, W2, b2, Wf, bf,
                             a1, a2, Wm, bm))
    return _sem_call(hs[0], hs[1], hs[2], w_omega, b_omega, u_omega)


NW = 32
ROWS = NBATCH // NW
NB_NODES = 4000
E = 16384
RCH = 8
NCHUNK = ROWS // RCH


def _sc_body(feat_hbm, nids_hbm, bias_hbm, bn_hbm, adjs_hbm,
             xg_hbm, a_hbm, bg_hbm,
             idx_v, xrows, cidx, sbuf, dbuf, aslab, rowbuf, outbuf,
             sem_x, sem_r, sem_o0, sem_o1):
    wid = lax.axis_index("s") * 2 + lax.axis_index("c")
    base = wid * ROWS
    zeros16 = jnp.zeros((16,), jnp.float32)
    ones16 = jnp.ones((16,), jnp.float32)
    osems = [sem_o0, sem_o1]
    ohandles = [None, None]
    gchunk = 0
    for i in range(P):
        pltpu.sync_copy(nids_hbm.at[i, 0, pl.ds(base, ROWS)], idx_v)
        hx = pltpu.async_copy(feat_hbm.at[i].at[idx_v], xrows, sem_x)
        pltpu.sync_copy(adjs_hbm.at[2 * i, 0], sbuf)
        pltpu.sync_copy(adjs_hbm.at[2 * i + 1, 0], dbuf)

        @plsc.parallel_loop(0, ROWS * NBATCH // 16, unroll=8)
        def _(j):
            aslab[j // (NBATCH // 16), pl.ds((j % (NBATCH // 16)) * 16, 16)] \
                = zeros16

        def edge_body(k, c):
            s = sbuf[pl.ds(k * 16, 16)]
            d = dbuf[pl.ds(k * 16, 16)]
            ld = d - base
            m = (ld >= 0) & (ld < ROWS)
            ld0 = jnp.where(m, ld, 0)
            plsc.addupdate_scatter(aslab, [ld0, s], ones16, mask=m)
            return c
        lax.fori_loop(0, E // 16, edge_body, 0)
        pltpu.sync_copy(aslab, a_hbm.at[i, pl.ds(base, ROWS)])
        hx.wait()
        pltpu.sync_copy(xrows, xg_hbm.at[i, pl.ds(base, ROWS)])
        if bias_hbm is None:
            continue
        pltpu.sync_copy(bn_hbm.at[i, 0], cidx)
        for rc in range(NCHUNK):
            cur = gchunk & 1
            hr = pltpu.async_copy(
                bias_hbm.at[i].at[cidx.at[pl.ds(base + rc * RCH, RCH)]],
                rowbuf, sem_r)
            hr.wait()
            if ohandles[cur] is not None:
                ohandles[cur].wait()

            @plsc.parallel_loop(0, RCH * NBATCH // 16, unroll=8)
            def _(t):
                r = t // (NBATCH // 16)
                j = t % (NBATCH // 16)
                ci = cidx[pl.ds(j * 16, 16)]
                vals = plsc.load_gather(
                    rowbuf, [jnp.full((16,), r, jnp.int32), ci])
                outbuf[cur, r, pl.ds(j * 16, 16)] = vals

            ohandles[cur] = pltpu.async_copy(
                outbuf.at[cur],
                bg_hbm.at[i, pl.ds(base + rc * RCH, RCH)],
                osems[cur])
            gchunk += 1
    for h in ohandles:
        if h is not None:
            h.wait()


@jax.jit
def _sc_call(features_list, n_ids, biases_mat_list, batch_node_list, adjs):
    mesh = plsc.VectorSubcoreMesh(core_axis_name="c", subcore_axis_name="s")
    f = pl.kernel(
        _sc_body,
        out_type=(
            jax.ShapeDtypeStruct((P, NBATCH, F), jnp.float32),
            jax.ShapeDtypeStruct((P, NBATCH, NBATCH), jnp.float32),
            jax.ShapeDtypeStruct((P, NBATCH, NBATCH), jnp.float32),
        ),
        mesh=mesh,
        compiler_params=pltpu.CompilerParams(use_tc_tiling_on_sc=False, needs_layout_passes=False),
        scratch_types=[
            pltpu.VMEM((ROWS,), jnp.int32),
            pltpu.VMEM((ROWS, F), jnp.float32),
            pltpu.VMEM((NBATCH,), jnp.int32),
            pltpu.VMEM((E,), jnp.int32),
            pltpu.VMEM((E,), jnp.int32),
            pltpu.VMEM((ROWS, NBATCH), jnp.float32),
            pltpu.VMEM((RCH, NB_NODES), jnp.float32),
            pltpu.VMEM((2, RCH, NBATCH), jnp.float32),
            pltpu.SemaphoreType.DMA,
            pltpu.SemaphoreType.DMA,
            pltpu.SemaphoreType.DMA,
            pltpu.SemaphoreType.DMA,
        ],
    )
    return f(features_list,
             n_ids.astype(jnp.int32).reshape(P, 1, NBATCH),
             biases_mat_list,
             batch_node_list.astype(jnp.int32).reshape(P, 1, NBATCH),
             adjs.astype(jnp.int32).reshape(2 * P, 1, E))


def _sc_colg_body(rows_hbm, bn_hbm, bg_hbm, cidx, rowbuf, outbuf,
                  sem_r0, sem_r1, sem_o0, sem_o1):
    wid = lax.axis_index("s") * 2 + lax.axis_index("c")
    base = wid * ROWS
    rsems = [sem_r0, sem_r1]
    osems = [sem_o0, sem_o1]
    ohandles = [None, None]
    rh = None
    gchunk = 0
    for i in range(P):
        pltpu.sync_copy(bn_hbm.at[i, 0], cidx)
        for rc in range(NCHUNK):
            cur = gchunk & 1
            if rh is None:
                rh = pltpu.async_copy(
                    rows_hbm.at[i, pl.ds(base + rc * RCH, RCH)],
                    rowbuf.at[cur], rsems[cur])
            rh.wait()
            nxt_i, nxt_rc = (i, rc + 1) if rc + 1 < NCHUNK else (i + 1, 0)
            if nxt_i < P:
                rh = pltpu.async_copy(
                    rows_hbm.at[nxt_i, pl.ds(base + nxt_rc * RCH, RCH)],
                    rowbuf.at[1 - cur], rsems[1 - cur])
            else:
                rh = None
            if ohandles[cur] is not None:
                ohandles[cur].wait()

            @plsc.parallel_loop(0, RCH * NBATCH // 16, unroll=8)
            def _(t):
                r = t // (NBATCH // 16)
                j = t % (NBATCH // 16)
                ci = cidx[pl.ds(j * 16, 16)]
                vals = plsc.load_gather(
                    rowbuf, [jnp.full((16,), cur, jnp.int32),
                             jnp.full((16,), r, jnp.int32), ci])
                outbuf[cur, r, pl.ds(j * 16, 16)] = vals

            ohandles[cur] = pltpu.async_copy(
                outbuf.at[cur],
                bg_hbm.at[i, pl.ds(base + rc * RCH, RCH)],
                osems[cur])
            gchunk += 1
    for h in ohandles:
        if h is not None:
            h.wait()


@jax.jit
def _sc_colg(rows, batch_node_list):
    mesh = plsc.VectorSubcoreMesh(core_axis_name="c", subcore_axis_name="s")
    f = pl.kernel(
        _sc_colg_body,
        out_type=jax.ShapeDtypeStruct((P, NBATCH, NBATCH), jnp.float32),
        mesh=mesh,
        compiler_params=pltpu.CompilerParams(use_tc_tiling_on_sc=False,
                                             needs_layout_passes=False),
        scratch_types=[
            pltpu.VMEM((NBATCH,), jnp.int32),
            pltpu.VMEM((2, RCH, NB_NODES), jnp.float32),
            pltpu.VMEM((2, RCH, NBATCH), jnp.float32),
            pltpu.SemaphoreType.DMA,
            pltpu.SemaphoreType.DMA,
            pltpu.SemaphoreType.DMA,
            pltpu.SemaphoreType.DMA,
        ],
    )
    return f(rows, batch_node_list.astype(jnp.int32).reshape(P, 1, NBATCH))


def _sc_body_noB(feat_hbm, nids_hbm, adjs_hbm, xg_hbm, a_hbm,
                 idx_v, xrows, sbuf, dbuf, aslab, sem_x):
    _sc_body(feat_hbm, nids_hbm, None, None, adjs_hbm, xg_hbm, a_hbm, None,
             idx_v, xrows, None, sbuf, dbuf, aslab, None, None,
             sem_x, None, None, None)


@jax.jit
def _sc_call_noB(features_list, n_ids, adjs):
    mesh = plsc.VectorSubcoreMesh(core_axis_name="c", subcore_axis_name="s")
    f = pl.kernel(
        _sc_body_noB,
        out_type=(
            jax.ShapeDtypeStruct((P, NBATCH, F), jnp.float32),
            jax.ShapeDtypeStruct((P, NBATCH, NBATCH), jnp.float32),
        ),
        mesh=mesh,
        compiler_params=pltpu.CompilerParams(use_tc_tiling_on_sc=False,
                                             needs_layout_passes=False),
        scratch_types=[
            pltpu.VMEM((ROWS,), jnp.int32),
            pltpu.VMEM((ROWS, F), jnp.float32),
            pltpu.VMEM((E,), jnp.int32),
            pltpu.VMEM((E,), jnp.int32),
            pltpu.VMEM((ROWS, NBATCH), jnp.float32),
            pltpu.SemaphoreType.DMA,
        ],
    )
    return f(features_list,
             n_ids.astype(jnp.int32).reshape(P, 1, NBATCH),
             adjs.astype(jnp.int32).reshape(2 * P, 1, E))


def kernel(features_list, biases_mat_list, batch_node_list, adjs, n_ids,
           device, RL_thresholds, W1, b1, W2, b2, Wf, bf, a1, a2, Wm, bm,
           w_omega, b_omega, u_omega):
    rows = jnp.take_along_axis(biases_mat_list, batch_node_list[:, :, None],
                               axis=1)
    xg, A = _sc_call_noB(features_list, n_ids, adjs)
    bias = _sc_colg(rows, batch_node_list)
    return _dense_call(xg, A, bias, W1, b1, W2, b2, Wf, bf, a1, a2,
                       Wm, bm, w_omega, b_omega, u_omega)


# speedup vs baseline: 1.1593x; 1.1593x over previous
import functools

import jax
import jax.numpy as jnp
from jax import lax
from jax.experimental import pallas as pl
from jax.experimental.pallas import tpu as pltpu
from jax.experimental.pallas import tpu_sc as plsc

P = 3
NBATCH = 1024
F = 128
NH = 8
HEAD_IN = F // NH
OUT_DIM = 64
OUT_SZ = OUT_DIM // NH
HID = 128


def _attn_body(xg_ref, A_ref, bias_ref, W1_ref, b1_ref, W2_ref, b2_ref,
               Wf_ref, bf_ref, a1_ref, a2_ref, Wm_ref, bm_ref,
               out_ref):
    x = xg_ref[0]
    A = A_ref[0]
    bias = bias_ref[0]

    deg = jnp.maximum(jnp.sum(A, axis=1, keepdims=True), 1.0)
    agg1 = jnp.dot(A, x, preferred_element_type=jnp.float32) / deg
    h = jax.nn.relu(jnp.dot(agg1, W1_ref[0], preferred_element_type=jnp.float32)
                    + b1_ref[0])
    agg2 = jnp.dot(A, h, preferred_element_type=jnp.float32) / deg
    fe = jnp.dot(agg2, W2_ref[0], preferred_element_type=jnp.float32) + b2_ref[0]

    attns = []
    for nh in range(NH):
        xh = fe[:, nh * HEAD_IN:(nh + 1) * HEAD_IN]
        f = jnp.dot(xh, Wf_ref[0, nh], preferred_element_type=jnp.float32) \
            + bf_ref[0, nh]
        f1 = jnp.dot(f, a1_ref[0, nh].reshape(OUT_SZ, 1),
                     preferred_element_type=jnp.float32)
        f2 = jnp.dot(f, a2_ref[0, nh].reshape(OUT_SZ, 1),
                     preferred_element_type=jnp.float32)
        logits = f1 + f2.T
        z = jnp.where(logits >= 0.0, logits, 0.2 * logits) + bias
        m = jnp.max(z, axis=1, keepdims=True)
        e = jnp.exp(z - m)
        s = jnp.sum(e, axis=1, keepdims=True)
        coefs = e / s
        av = jnp.dot(coefs, f, preferred_element_type=jnp.float32)
        attns.append(jnp.where(av > 0.0, av, jnp.exp(av) - 1.0))
    h_1 = jnp.concatenate(attns, axis=-1)
    h1t = jnp.dot(h_1, Wm_ref[...], preferred_element_type=jnp.float32) \
        + bm_ref[...]
    out_ref[...] = h1t


def _attn_call(i, xg, A, bias, W1, b1, W2, b2, Wf, bf, a1, a2, Wm, bm):
    bs_meta3 = lambda shp: pl.BlockSpec((1,) + shp,
                                        lambda g, i=i: (i,) + (0,) * len(shp))
    bs_full = lambda shp: pl.BlockSpec(shp, lambda g: (0,) * len(shp))
    return pl.pallas_call(
        _attn_body,
        grid=(1,),
        in_specs=[
            bs_meta3((NBATCH, F)),
            bs_meta3((NBATCH, NBATCH)),
            bs_meta3((NBATCH, NBATCH)),
            bs_meta3((F, HID)),
            bs_meta3((1, HID)),
            bs_meta3((HID, F)),
            bs_meta3((1, F)),
            bs_meta3((NH, HEAD_IN, OUT_SZ)),
            bs_meta3((NH, 1, OUT_SZ)),
            bs_meta3((NH, OUT_SZ)),
            bs_meta3((NH, OUT_SZ)),
            bs_full((OUT_DIM, OUT_DIM)),
            bs_full((1, OUT_DIM)),
        ],
        out_specs=pl.BlockSpec((NBATCH, OUT_DIM), lambda g: (0, 0)),
        out_shape=jax.ShapeDtypeStruct((NBATCH, OUT_DIM), jnp.float32),
    )(xg, A, bias, W1, b1[:, None, :], W2, b2[:, None, :], Wf,
      bf[:, :, None, :], a1, a2, Wm, bm[None])


def _sem_body(m0_ref, m1_ref, m2_ref, wom_ref, bom_ref, uom_ref, out_ref):
    ms = [m0_ref[...], m1_ref[...], m2_ref[...]]
    vus = []
    for j in range(P):
        v = jnp.tanh(jnp.dot(ms[j], wom_ref[...],
                             preferred_element_type=jnp.float32)
                     + bom_ref[...])
        vu = jnp.dot(v, uom_ref[...].reshape(HID, 1),
                     preferred_element_type=jnp.float32)
        vus.append(vu)
    vu_all = jnp.concatenate(vus, axis=-1)
    mx = jnp.max(vu_all, axis=1, keepdims=True)
    ev = jnp.exp(vu_all - mx)
    al = ev / jnp.sum(ev, axis=1, keepdims=True)
    acc = al[:, 0:1] * ms[0]
    for j in range(1, P):
        acc = acc + al[:, j:j + 1] * ms[j]
    out_ref[...] = acc


def _sem_call(m0, m1, m2, w_omega, b_omega, u_omega):
    return pl.pallas_call(
        _sem_body,
        out_shape=jax.ShapeDtypeStruct((NBATCH, OUT_DIM), jnp.float32),
    )(m0, m1, m2, w_omega, b_omega[None], u_omega[None])


@jax.jit
def _dense_call(xg, A, bias, W1, b1, W2, b2,
                Wf, bf, a1, a2, Wm, bm, w_omega, b_omega, u_omega):
    hs = []
    for i in range(P):
        hs.append(_attn_call(i, xg, A, bias, W1, b1, W2, b2, Wf, bf,
                             a1, a2, Wm, bm))
    return _sem_call(hs[0], hs[1], hs[2], w_omega, b_omega, u_omega)


NW = 32
ROWS = NBATCH // NW
NB_NODES = 4000
E = 16384
RCH = 8
NCHUNK = ROWS // RCH


def _sc_body(feat_hbm, nids_hbm, bias_hbm, bn_hbm, adjs_hbm,
             xg_hbm, a_hbm, bg_hbm,
             idx_v, xrows, cidx, sbuf, dbuf, aslab, rowbuf, outbuf,
             sem_x, sem_r, sem_o0, sem_o1):
    wid = lax.axis_index("s") * 2 + lax.axis_index("c")
    base = wid * ROWS
    zeros16 = jnp.zeros((16,), jnp.float32)
    ones16 = jnp.ones((16,), jnp.float32)
    osems = [sem_o0, sem_o1]
    ohandles = [None, None]
    gchunk = 0
    for i in range(P):
        pltpu.sync_copy(nids_hbm.at[i, 0, pl.ds(base, ROWS)], idx_v)
        hx = pltpu.async_copy(feat_hbm.at[i].at[idx_v], xrows, sem_x)
        pltpu.sync_copy(adjs_hbm.at[2 * i, 0], sbuf)
        pltpu.sync_copy(adjs_hbm.at[2 * i + 1, 0], dbuf)

        @plsc.parallel_loop(0, ROWS * NBATCH // 16, unroll=8)
        def _(j):
            aslab[j // (NBATCH // 16), pl.ds((j % (NBATCH // 16)) * 16, 16)] \
                = zeros16

        def edge_body(k, c):
            s = sbuf[pl.ds(k * 16, 16)]
            d = dbuf[pl.ds(k * 16, 16)]
            ld = d - base
            m = (ld >= 0) & (ld < ROWS)
            ld0 = jnp.where(m, ld, 0)
            plsc.addupdate_scatter(aslab, [ld0, s], ones16, mask=m)
            return c
        lax.fori_loop(0, E // 16, edge_body, 0)
        pltpu.sync_copy(aslab, a_hbm.at[i, pl.ds(base, ROWS)])
        hx.wait()
        pltpu.sync_copy(xrows, xg_hbm.at[i, pl.ds(base, ROWS)])
        if bias_hbm is None:
            continue
        pltpu.sync_copy(bn_hbm.at[i, 0], cidx)
        for rc in range(NCHUNK):
            cur = gchunk & 1
            hr = pltpu.async_copy(
                bias_hbm.at[i].at[cidx.at[pl.ds(base + rc * RCH, RCH)]],
                rowbuf, sem_r)
            hr.wait()
            if ohandles[cur] is not None:
                ohandles[cur].wait()

            @plsc.parallel_loop(0, RCH * NBATCH // 16, unroll=8)
            def _(t):
                r = t // (NBATCH // 16)
                j = t % (NBATCH // 16)
                ci = cidx[pl.ds(j * 16, 16)]
                vals = plsc.load_gather(
                    rowbuf, [jnp.full((16,), r, jnp.int32), ci])
                outbuf[cur, r, pl.ds(j * 16, 16)] = vals

            ohandles[cur] = pltpu.async_copy(
                outbuf.at[cur],
                bg_hbm.at[i, pl.ds(base + rc * RCH, RCH)],
                osems[cur])
            gchunk += 1
    for h in ohandles:
        if h is not None:
            h.wait()


@jax.jit
def _sc_call(features_list, n_ids, biases_mat_list, batch_node_list, adjs):
    mesh = plsc.VectorSubcoreMesh(core_axis_name="c", subcore_axis_name="s")
    f = pl.kernel(
        _sc_body,
        out_type=(
            jax.ShapeDtypeStruct((P, NBATCH, F), jnp.float32),
            jax.ShapeDtypeStruct((P, NBATCH, NBATCH), jnp.float32),
            jax.ShapeDtypeStruct((P, NBATCH, NBATCH), jnp.float32),
        ),
        mesh=mesh,
        compiler_params=pltpu.CompilerParams(use_tc_tiling_on_sc=False, needs_layout_passes=False),
        scratch_types=[
            pltpu.VMEM((ROWS,), jnp.int32),
            pltpu.VMEM((ROWS, F), jnp.float32),
            pltpu.VMEM((NBATCH,), jnp.int32),
            pltpu.VMEM((E,), jnp.int32),
            pltpu.VMEM((E,), jnp.int32),
            pltpu.VMEM((ROWS, NBATCH), jnp.float32),
            pltpu.VMEM((RCH, NB_NODES), jnp.float32),
            pltpu.VMEM((2, RCH, NBATCH), jnp.float32),
            pltpu.SemaphoreType.DMA,
            pltpu.SemaphoreType.DMA,
            pltpu.SemaphoreType.DMA,
            pltpu.SemaphoreType.DMA,
        ],
    )
    return f(features_list,
             n_ids.astype(jnp.int32).reshape(P, 1, NBATCH),
             biases_mat_list,
             batch_node_list.astype(jnp.int32).reshape(P, 1, NBATCH),
             adjs.astype(jnp.int32).reshape(2 * P, 1, E))


def _sc_colg_body(rows_hbm, bn_hbm, bg_hbm, cidx, rowbuf, outbuf,
                  sem_r0, sem_r1, sem_o0, sem_o1):
    wid = lax.axis_index("s") * 2 + lax.axis_index("c")
    base = wid * ROWS
    rsems = [sem_r0, sem_r1]
    osems = [sem_o0, sem_o1]
    ohandles = [None, None]
    rh = None
    gchunk = 0
    for i in range(P):
        pltpu.sync_copy(bn_hbm.at[i, 0], cidx)
        for rc in range(NCHUNK):
            cur = gchunk & 1
            if rh is None:
                rh = pltpu.async_copy(
                    rows_hbm.at[i, pl.ds(base + rc * RCH, RCH)],
                    rowbuf.at[cur], rsems[cur])
            rh.wait()
            nxt_i, nxt_rc = (i, rc + 1) if rc + 1 < NCHUNK else (i + 1, 0)
            if nxt_i < P:
                rh = pltpu.async_copy(
                    rows_hbm.at[nxt_i, pl.ds(base + nxt_rc * RCH, RCH)],
                    rowbuf.at[1 - cur], rsems[1 - cur])
            else:
                rh = None
            if ohandles[cur] is not None:
                ohandles[cur].wait()

            @plsc.parallel_loop(0, RCH * NBATCH // 16, unroll=8)
            def _(t):
                r = t // (NBATCH // 16)
                j = t % (NBATCH // 16)
                ci = cidx[pl.ds(j * 16, 16)]
                vals = plsc.load_gather(
                    rowbuf, [jnp.full((16,), cur, jnp.int32),
                             jnp.full((16,), r, jnp.int32), ci])
                outbuf[cur, r, pl.ds(j * 16, 16)] = vals

            ohandles[cur] = pltpu.async_copy(
                outbuf.at[cur],
                bg_hbm.at[i, pl.ds(base + rc * RCH, RCH)],
                osems[cur])
            gchunk += 1
    for h in ohandles:
        if h is not None:
            h.wait()


@jax.jit
def _sc_colg(rows, batch_node_list):
    mesh = plsc.VectorSubcoreMesh(core_axis_name="c", subcore_axis_name="s")
    f = pl.kernel(
        _sc_colg_body,
        out_type=jax.ShapeDtypeStruct((P, NBATCH, NBATCH), jnp.float32),
        mesh=mesh,
        compiler_params=pltpu.CompilerParams(use_tc_tiling_on_sc=True,
                                             needs_layout_passes=False),
        scratch_types=[
            pltpu.VMEM((NBATCH,), jnp.int32),
            pltpu.VMEM((2, RCH, NB_NODES), jnp.float32),
            pltpu.VMEM((2, RCH, NBATCH), jnp.float32),
            pltpu.SemaphoreType.DMA,
            pltpu.SemaphoreType.DMA,
            pltpu.SemaphoreType.DMA,
            pltpu.SemaphoreType.DMA,
        ],
    )
    return f(rows, batch_node_list.astype(jnp.int32).reshape(P, 1, NBATCH))


def _sc_body_noB(feat_hbm, nids_hbm, adjs_hbm, xg_hbm, a_hbm,
                 idx_v, xrows, sbuf, dbuf, aslab, sem_x):
    _sc_body(feat_hbm, nids_hbm, None, None, adjs_hbm, xg_hbm, a_hbm, None,
             idx_v, xrows, None, sbuf, dbuf, aslab, None, None,
             sem_x, None, None, None)


@jax.jit
def _sc_call_noB(features_list, n_ids, adjs):
    mesh = plsc.VectorSubcoreMesh(core_axis_name="c", subcore_axis_name="s")
    f = pl.kernel(
        _sc_body_noB,
        out_type=(
            jax.ShapeDtypeStruct((P, NBATCH, F), jnp.float32),
            jax.ShapeDtypeStruct((P, NBATCH, NBATCH), jnp.float32),
        ),
        mesh=mesh,
        compiler_params=pltpu.CompilerParams(use_tc_tiling_on_sc=False,
                                             needs_layout_passes=False),
        scratch_types=[
            pltpu.VMEM((ROWS,), jnp.int32),
            pltpu.VMEM((ROWS, F), jnp.float32),
            pltpu.VMEM((E,), jnp.int32),
            pltpu.VMEM((E,), jnp.int32),
            pltpu.VMEM((ROWS, NBATCH), jnp.float32),
            pltpu.SemaphoreType.DMA,
        ],
    )
    return f(features_list,
             n_ids.astype(jnp.int32).reshape(P, 1, NBATCH),
             adjs.astype(jnp.int32).reshape(2 * P, 1, E))


def kernel(features_list, biases_mat_list, batch_node_list, adjs, n_ids,
           device, RL_thresholds, W1, b1, W2, b2, Wf, bf, a1, a2, Wm, bm,
           w_omega, b_omega, u_omega):
    rows = jnp.take_along_axis(biases_mat_list, batch_node_list[:, :, None],
                               axis=1)
    xg, A = _sc_call_noB(features_list, n_ids, adjs)
    bias = _sc_colg(rows, batch_node_list)
    return _dense_call(xg, A, bias, W1, b1, W2, b2, Wf, bf, a1, a2,
                       Wm, bm, w_omega, b_omega, u_omega)


# speedup vs baseline: 1.1596x; 1.0002x over previous
import functools

import jax
import jax.numpy as jnp
from jax import lax
from jax.experimental import pallas as pl
from jax.experimental.pallas import tpu as pltpu
from jax.experimental.pallas import tpu_sc as plsc

P = 3
NBATCH = 1024
F = 128
NH = 8
HEAD_IN = F // NH
OUT_DIM = 64
OUT_SZ = OUT_DIM // NH
HID = 128


def _attn_body(xg_ref, A_ref, bias_ref, W1_ref, b1_ref, W2_ref, b2_ref,
               Wf_ref, bf_ref, a1_ref, a2_ref, Wm_ref, bm_ref,
               out_ref):
    x = xg_ref[0]
    A = A_ref[0]
    bias = bias_ref[0]

    deg = jnp.maximum(jnp.sum(A, axis=1, keepdims=True), 1.0)
    agg1 = jnp.dot(A, x, preferred_element_type=jnp.float32) / deg
    h = jax.nn.relu(jnp.dot(agg1, W1_ref[0], preferred_element_type=jnp.float32)
                    + b1_ref[0])
    agg2 = jnp.dot(A, h, preferred_element_type=jnp.float32) / deg
    fe = jnp.dot(agg2, W2_ref[0], preferred_element_type=jnp.float32) + b2_ref[0]

    attns = []
    for nh in range(NH):
        xh = fe[:, nh * HEAD_IN:(nh + 1) * HEAD_IN]
        f = jnp.dot(xh, Wf_ref[0, nh], preferred_element_type=jnp.float32) \
            + bf_ref[0, nh]
        f1 = jnp.dot(f, a1_ref[0, nh].reshape(OUT_SZ, 1),
                     preferred_element_type=jnp.float32)
        f2 = jnp.dot(f, a2_ref[0, nh].reshape(OUT_SZ, 1),
                     preferred_element_type=jnp.float32)
        logits = f1 + f2.T
        z = jnp.where(logits >= 0.0, logits, 0.2 * logits) + bias
        m = jnp.max(z, axis=1, keepdims=True)
        e = jnp.exp(z - m)
        s = jnp.sum(e, axis=1, keepdims=True)
        coefs = e / s
        av = jnp.dot(coefs, f, preferred_element_type=jnp.float32)
        attns.append(jnp.where(av > 0.0, av, jnp.exp(av) - 1.0))
    h_1 = jnp.concatenate(attns, axis=-1)
    h1t = jnp.dot(h_1, Wm_ref[...], preferred_element_type=jnp.float32) \
        + bm_ref[...]
    out_ref[...] = h1t


def _attn_call(i, xg, A, bias, W1, b1, W2, b2, Wf, bf, a1, a2, Wm, bm):
    bs_meta3 = lambda shp: pl.BlockSpec((1,) + shp,
                                        lambda g, i=i: (i,) + (0,) * len(shp))
    bs_full = lambda shp: pl.BlockSpec(shp, lambda g: (0,) * len(shp))
    return pl.pallas_call(
        _attn_body,
        grid=(1,),
        in_specs=[
            bs_meta3((NBATCH, F)),
            bs_meta3((NBATCH, NBATCH)),
            bs_meta3((NBATCH, NBATCH)),
            bs_meta3((F, HID)),
            bs_meta3((1, HID)),
            bs_meta3((HID, F)),
            bs_meta3((1, F)),
            bs_meta3((NH, HEAD_IN, OUT_SZ)),
            bs_meta3((NH, 1, OUT_SZ)),
            bs_meta3((NH, OUT_SZ)),
            bs_meta3((NH, OUT_SZ)),
            bs_full((OUT_DIM, OUT_DIM)),
            bs_full((1, OUT_DIM)),
        ],
        out_specs=pl.BlockSpec((NBATCH, OUT_DIM), lambda g: (0, 0)),
        out_shape=jax.ShapeDtypeStruct((NBATCH, OUT_DIM), jnp.float32),
    )(xg, A, bias, W1, b1[:, None, :], W2, b2[:, None, :], Wf,
      bf[:, :, None, :], a1, a2, Wm, bm[None])


def _sem_body(m0_ref, m1_ref, m2_ref, wom_ref, bom_ref, uom_ref, out_ref):
    ms = [m0_ref[...], m1_ref[...], m2_ref[...]]
    vus = []
    for j in range(P):
        v = jnp.tanh(jnp.dot(ms[j], wom_ref[...],
                             preferred_element_type=jnp.float32)
                     + bom_ref[...])
        vu = jnp.dot(v, uom_ref[...].reshape(HID, 1),
                     preferred_element_type=jnp.float32)
        vus.append(vu)
    vu_all = jnp.concatenate(vus, axis=-1)
    mx = jnp.max(vu_all, axis=1, keepdims=True)
    ev = jnp.exp(vu_all - mx)
    al = ev / jnp.sum(ev, axis=1, keepdims=True)
    acc = al[:, 0:1] * ms[0]
    for j in range(1, P):
        acc = acc + al[:, j:j + 1] * ms[j]
    out_ref[...] = acc


def _sem_call(m0, m1, m2, w_omega, b_omega, u_omega):
    return pl.pallas_call(
        _sem_body,
        out_shape=jax.ShapeDtypeStruct((NBATCH, OUT_DIM), jnp.float32),
    )(m0, m1, m2, w_omega, b_omega[None], u_omega[None])


@jax.jit
def _dense_call(xg, A, bias, W1, b1, W2, b2,
                Wf, bf, a1, a2, Wm, bm, w_omega, b_omega, u_omega):
    hs = []
    for i in range(P):
        hs.append(_attn_call(i, xg, A, bias, W1, b1, W2, b2, Wf, bf,
                             a1, a2, Wm, bm))
    return _sem_call(hs[0], hs[1], hs[2], w_omega, b_omega, u_omega)


NW = 32
ROWS = NBATCH // NW
NB_NODES = 4000
E = 16384
RCH = 8
NCHUNK = ROWS // RCH


def _sc_body(feat_hbm, nids_hbm, bias_hbm, bn_hbm, adjs_hbm,
             xg_hbm, a_hbm, bg_hbm,
             idx_v, xrows, cidx, sbuf, dbuf, aslab, rowbuf, outbuf,
             sem_x, sem_r, sem_o0, sem_o1):
    wid = lax.axis_index("s") * 2 + lax.axis_index("c")
    base = wid * ROWS
    zeros16 = jnp.zeros((16,), jnp.float32)
    ones16 = jnp.ones((16,), jnp.float32)
    osems = [sem_o0, sem_o1]
    ohandles = [None, None]
    gchunk = 0
    for i in range(P):
        pltpu.sync_copy(nids_hbm.at[i, 0, pl.ds(base, ROWS)], idx_v)
        hx = pltpu.async_copy(feat_hbm.at[i].at[idx_v], xrows, sem_x)
        pltpu.sync_copy(adjs_hbm.at[2 * i, 0], sbuf)
        pltpu.sync_copy(adjs_hbm.at[2 * i + 1, 0], dbuf)

        @plsc.parallel_loop(0, ROWS * NBATCH // 16, unroll=8)
        def _(j):
            aslab[j // (NBATCH // 16), pl.ds((j % (NBATCH // 16)) * 16, 16)] \
                = zeros16

        def edge_body(k, c):
            s = sbuf[pl.ds(k * 16, 16)]
            d = dbuf[pl.ds(k * 16, 16)]
            ld = d - base
            m = (ld >= 0) & (ld < ROWS)
            ld0 = jnp.where(m, ld, 0)
            plsc.addupdate_scatter(aslab, [ld0, s], ones16, mask=m)
            return c
        lax.fori_loop(0, E // 16, edge_body, 0)
        pltpu.sync_copy(aslab, a_hbm.at[i, pl.ds(base, ROWS)])
        hx.wait()
        pltpu.sync_copy(xrows, xg_hbm.at[i, pl.ds(base, ROWS)])
        if bias_hbm is None:
            continue
        pltpu.sync_copy(bn_hbm.at[i, 0], cidx)
        for rc in range(NCHUNK):
            cur = gchunk & 1
            hr = pltpu.async_copy(
                bias_hbm.at[i].at[cidx.at[pl.ds(base + rc * RCH, RCH)]],
                rowbuf, sem_r)
            hr.wait()
            if ohandles[cur] is not None:
                ohandles[cur].wait()

            @plsc.parallel_loop(0, RCH * NBATCH // 16, unroll=8)
            def _(t):
                r = t // (NBATCH // 16)
                j = t % (NBATCH // 16)
                ci = cidx[pl.ds(j * 16, 16)]
                vals = plsc.load_gather(
                    rowbuf, [jnp.full((16,), r, jnp.int32), ci])
                outbuf[cur, r, pl.ds(j * 16, 16)] = vals

            ohandles[cur] = pltpu.async_copy(
                outbuf.at[cur],
                bg_hbm.at[i, pl.ds(base + rc * RCH, RCH)],
                osems[cur])
            gchunk += 1
    for h in ohandles:
        if h is not None:
            h.wait()


@jax.jit
def _sc_call(features_list, n_ids, biases_mat_list, batch_node_list, adjs):
    mesh = plsc.VectorSubcoreMesh(core_axis_name="c", subcore_axis_name="s")
    f = pl.kernel(
        _sc_body,
        out_type=(
            jax.ShapeDtypeStruct((P, NBATCH, F), jnp.float32),
            jax.ShapeDtypeStruct((P, NBATCH, NBATCH), jnp.float32),
            jax.ShapeDtypeStruct((P, NBATCH, NBATCH), jnp.float32),
        ),
        mesh=mesh,
        compiler_params=pltpu.CompilerParams(use_tc_tiling_on_sc=False, needs_layout_passes=False),
        scratch_types=[
            pltpu.VMEM((ROWS,), jnp.int32),
            pltpu.VMEM((ROWS, F), jnp.float32),
            pltpu.VMEM((NBATCH,), jnp.int32),
            pltpu.VMEM((E,), jnp.int32),
            pltpu.VMEM((E,), jnp.int32),
            pltpu.VMEM((ROWS, NBATCH), jnp.float32),
            pltpu.VMEM((RCH, NB_NODES), jnp.float32),
            pltpu.VMEM((2, RCH, NBATCH), jnp.float32),
            pltpu.SemaphoreType.DMA,
            pltpu.SemaphoreType.DMA,
            pltpu.SemaphoreType.DMA,
            pltpu.SemaphoreType.DMA,
        ],
    )
    return f(features_list,
             n_ids.astype(jnp.int32).reshape(P, 1, NBATCH),
             biases_mat_list,
             batch_node_list.astype(jnp.int32).reshape(P, 1, NBATCH),
             adjs.astype(jnp.int32).reshape(2 * P, 1, E))


def _sc_colg_body(rows_hbm, bn_hbm, bg_hbm, cidx, rowbuf, outbuf,
                  sem_r0, sem_r1, sem_o0, sem_o1):
    wid = lax.axis_index("s") * 2 + lax.axis_index("c")
    base = wid * ROWS
    rsems = [sem_r0, sem_r1]
    osems = [sem_o0, sem_o1]
    ohandles = [None, None]
    rh = None
    gchunk = 0
    for i in range(P):
        pltpu.sync_copy(bn_hbm.at[i, 0], cidx)
        for rc in range(NCHUNK):
            cur = gchunk & 1
            if rh is None:
                rh = pltpu.async_copy(
                    rows_hbm.at[i, pl.ds(base + rc * RCH, RCH)],
                    rowbuf.at[cur], rsems[cur])
            rh.wait()
            nxt_i, nxt_rc = (i, rc + 1) if rc + 1 < NCHUNK else (i + 1, 0)
            if nxt_i < P:
                rh = pltpu.async_copy(
                    rows_hbm.at[nxt_i, pl.ds(base + nxt_rc * RCH, RCH)],
                    rowbuf.at[1 - cur], rsems[1 - cur])
            else:
                rh = None
            if ohandles[cur] is not None:
                ohandles[cur].wait()

            @plsc.parallel_loop(0, RCH * NBATCH // 16, unroll=8)
            def _(t):
                r = t // (NBATCH // 16)
                j = t % (NBATCH // 16)
                ci = cidx[pl.ds(j * 16, 16)]
                vals = plsc.load_gather(
                    rowbuf, [jnp.full((16,), cur, jnp.int32),
                             jnp.full((16,), r, jnp.int32), ci])
                outbuf[cur, r, pl.ds(j * 16, 16)] = vals

            ohandles[cur] = pltpu.async_copy(
                outbuf.at[cur],
                bg_hbm.at[i, pl.ds(base + rc * RCH, RCH)],
                osems[cur])
            gchunk += 1
    for h in ohandles:
        if h is not None:
            h.wait()


@jax.jit
def _sc_colg(rows, batch_node_list):
    mesh = plsc.VectorSubcoreMesh(core_axis_name="c", subcore_axis_name="s")
    f = pl.kernel(
        _sc_colg_body,
        out_type=jax.ShapeDtypeStruct((P, NBATCH, NBATCH), jnp.float32),
        mesh=mesh,
        compiler_params=pltpu.CompilerParams(use_tc_tiling_on_sc=True,
                                             needs_layout_passes=False),
        scratch_types=[
            pltpu.VMEM((NBATCH,), jnp.int32),
            pltpu.VMEM((2, RCH, NB_NODES), jnp.float32),
            pltpu.VMEM((2, RCH, NBATCH), jnp.float32),
            pltpu.SemaphoreType.DMA,
            pltpu.SemaphoreType.DMA,
            pltpu.SemaphoreType.DMA,
            pltpu.SemaphoreType.DMA,
        ],
    )
    return f(rows, batch_node_list.astype(jnp.int32).reshape(P, 1, NBATCH))


def _sc_body_noB(feat_hbm, nids_hbm, adjs_hbm, xg_hbm, a_hbm,
                 idx_v, xrows, sbuf, dbuf, aslab, sem_x):
    _sc_body(feat_hbm, nids_hbm, None, None, adjs_hbm, xg_hbm, a_hbm, None,
             idx_v, xrows, None, sbuf, dbuf, aslab, None, None,
             sem_x, None, None, None)


@jax.jit
def _sc_call_noB(features_list, n_ids, adjs):
    mesh = plsc.VectorSubcoreMesh(core_axis_name="c", subcore_axis_name="s")
    f = pl.kernel(
        _sc_body_noB,
        out_type=(
            jax.ShapeDtypeStruct((P, NBATCH, F), jnp.float32),
            jax.ShapeDtypeStruct((P, NBATCH, NBATCH), jnp.float32),
        ),
        mesh=mesh,
        compiler_params=pltpu.CompilerParams(use_tc_tiling_on_sc=True,
                                             needs_layout_passes=False),
        scratch_types=[
            pltpu.VMEM((ROWS,), jnp.int32),
            pltpu.VMEM((ROWS, F), jnp.float32),
            pltpu.VMEM((E,), jnp.int32),
            pltpu.VMEM((E,), jnp.int32),
            pltpu.VMEM((ROWS, NBATCH), jnp.float32),
            pltpu.SemaphoreType.DMA,
        ],
    )
    return f(features_list,
             n_ids.astype(jnp.int32).reshape(P, 1, NBATCH),
             adjs.astype(jnp.int32).reshape(2 * P, 1, E))


def kernel(features_list, biases_mat_list, batch_node_list, adjs, n_ids,
           device, RL_thresholds, W1, b1, W2, b2, Wf, bf, a1, a2, Wm, bm,
           w_omega, b_omega, u_omega):
    rows = jnp.take_along_axis(biases_mat_list, batch_node_list[:, :, None],
                               axis=1)
    xg, A = _sc_call_noB(features_list, n_ids, adjs)
    bias = _sc_colg(rows, batch_node_list)
    return _dense_call(xg, A, bias, W1, b1, W2, b2, Wf, bf, a1, a2,
                       Wm, bm, w_omega, b_omega, u_omega)
